# baseline (speedup 1.0000x reference)
import functools

import jax
import jax.numpy as jnp
import numpy as np
from jax import lax
from jax.experimental import pallas as pl
from jax.experimental.pallas import tpu as pltpu
from jax.experimental.pallas import tpu_sc as plsc

NC = 2
NS = 16
NW = NC * NS
L = 16
LN_EPS = 1e-5
SEQ_PER_CHUNK = 2
NBUF = 4


def _rsqrt_newton(x):
    i = lax.bitcast_convert_type(x, jnp.int32)
    i = jnp.int32(0x5F3759DF) - lax.shift_right_arithmetic(i, jnp.int32(1))
    y = lax.bitcast_convert_type(i, jnp.float32)
    half_x = jnp.float32(0.5) * x
    for _ in range(3):
        y = y * (jnp.float32(1.5) - half_x * y * y)
    return y


def _build(n_tot, S, emb, interpret=False):
    chunk = SEQ_PER_CHUNK * S
    rows_pw = n_tot // NW
    G = rows_pw // chunk
    cpad = ((chunk + 15) // 16) * 16
    FV = emb // L
    inv_emb = np.float32(1.0 / emb)

    mesh = plsc.VectorSubcoreMesh(
        core_axis_name="c", subcore_axis_name="s", num_cores=NC, num_subcores=NS
    )

    @functools.partial(
        pl.kernel,
        out_type=jax.ShapeDtypeStruct((n_tot, emb), jnp.float32),
        mesh=mesh,
        scratch_types=[
            pltpu.VMEM((G, chunk), jnp.int32),
            pltpu.VMEM((S, emb), jnp.float32),
            pltpu.VMEM((2, emb), jnp.float32),
            pltpu.VMEM((NBUF, cpad, emb), jnp.float32),
            pltpu.VMEM((NBUF, cpad, emb), jnp.float32),
            pltpu.VMEM((NBUF, 2, cpad), jnp.float32),
        ]
        + [pltpu.SemaphoreType.DMA] * (2 * NBUF),
        compiler_params=pltpu.CompilerParams(
            needs_layout_passes=False, use_tc_tiling_on_sc=False
        ),
        interpret=interpret,
    )
    def k(idx_hbm, table_hbm, pos_hbm, wb_hbm, out_hbm,
          idx_v, pos_v, wb_v, x_v, out_v, st_v, *sems):
        gsems = sems[:NBUF]
        osems = sems[NBUF:]
        wid = lax.axis_index("s") * NC + lax.axis_index("c")
        row0 = wid * rows_pw

        pltpu.sync_copy(idx_hbm.at[wid], idx_v)
        pltpu.sync_copy(pos_hbm, pos_v)
        pltpu.sync_copy(wb_hbm, wb_v)

        def gather_start(g, b):
            pltpu.async_copy(
                table_hbm.at[idx_v.at[g]], x_v.at[b, pl.ds(0, chunk)], gsems[b]
            )

        def gather_wait(b):
            pltpu.make_async_copy(
                table_hbm.at[idx_v.at[0]], x_v.at[b, pl.ds(0, chunk)], gsems[b]
            ).wait()

        def out_start(g, b):
            pltpu.async_copy(
                out_v.at[b, pl.ds(0, chunk)],
                out_hbm.at[pl.ds(row0 + g * chunk, chunk)],
                osems[b],
            )

        def out_wait(b):
            pltpu.make_async_copy(
                out_v.at[b, pl.ds(0, chunk)],
                out_hbm.at[pl.ds(0, chunk)],
                osems[b],
            ).wait()

        Ws = [wb_v[0, pl.ds(j * L, L)] for j in range(FV)]
        Bs = [wb_v[1, pl.ds(j * L, L)] for j in range(FV)]
        lane15 = lax.iota(jnp.int32, L) == jnp.int32(L - 1)

        def compute_chunk(b):
            @plsc.parallel_loop(0, S, unroll=2)
            def _(p):
                ps = [pos_v[p, pl.ds(j * L, L)] for j in range(FV)]
                for h in range(SEQ_PER_CHUNK):
                    i = h * S + p
                    xs = []
                    for j in range(FV):
                        xs.append(x_v[b, i, pl.ds(j * L, L)] + ps[j])
                    ssum = (xs[0] + xs[1]) + (xs[2] + xs[3])
                    qs = [x * x for x in xs]
                    qsum = (qs[0] + qs[1]) + (qs[2] + qs[3])
                    for j in range(FV):
                        x_v[b, i, pl.ds(j * L, L)] = xs[j]
                    sc = plsc.cumsum(ssum)
                    qc = plsc.cumsum(qsum)
                    iv = jnp.broadcast_to(i, (L,)).astype(jnp.int32)
                    plsc.store_scatter(st_v.at[b, 0], [iv], sc, mask=lane15)
                    plsc.store_scatter(st_v.at[b, 1], [iv], qc, mask=lane15)

            @plsc.parallel_loop(0, cpad // 16)
            def _(kg):
                r0 = kg * 16
                sv = st_v[b, 0, pl.ds(r0, L)]
                qv = st_v[b, 1, pl.ds(r0, L)]
                mean16 = sv * inv_emb
                var16 = qv * inv_emb - mean16 * mean16
                rstd16 = _rsqrt_newton(var16 + np.float32(LN_EPS))
                st_v[b, 0, pl.ds(r0, L)] = rstd16
                st_v[b, 1, pl.ds(r0, L)] = mean16 * rstd16

            @plsc.parallel_loop(0, chunk, unroll=4)
            def _(i):
                iv = jnp.broadcast_to(i, (L,)).astype(jnp.int32)
                rs = plsc.load_gather(st_v.at[b, 0], [iv])
                cm = plsc.load_gather(st_v.at[b, 1], [iv])
                for j in range(FV):
                    x = x_v[b, i, pl.ds(j * L, L)]
                    out_v[b, i, pl.ds(j * L, L)] = (x * rs - cm) * Ws[j] + Bs[j]

        for b0 in range(NBUF):
            gather_start(b0, b0)

        def ring_step(outer, _):
            for b in range(NBUF):
                g = outer * NBUF + b

                @pl.when(g < G)
                def _():
                    gather_wait(b)

                    @pl.when(g >= NBUF)
                    def _():
                        out_wait(b)

                    compute_chunk(b)

                    @pl.when(g + NBUF < G)
                    def _():
                        gather_start(g + NBUF, b)

                    out_start(g, b)
            return 0

        lax.fori_loop(0, (G + NBUF - 1) // NBUF, ring_step, 0)

        for b0 in range(NBUF):
            out_wait(b0)

    return k


@functools.lru_cache(maxsize=None)
def _kernel_fn(n_tot, S, emb, interpret):
    return _build(n_tot, S, emb, interpret)


def _call(src, emb_table, pos_table, ln_w, ln_b, interpret=False):
    N, S = src.shape
    emb = emb_table.shape[1]
    n_tot = N * S
    chunk = SEQ_PER_CHUNK * S
    assert n_tot % (NW * chunk) == 0
    G = n_tot // (NW * chunk)

    idx_r = src.reshape(NW, G, chunk).astype(jnp.int32)
    pos = pos_table[:S]
    wb = jnp.stack([ln_w, ln_b])
    fn = _kernel_fn(n_tot, S, emb, interpret)
    out = fn(idx_r, emb_table, pos, wb)
    return out.reshape(N, S, emb)


def kernel(src, emb_table, pos_table, ln_w, ln_b):
    return _call(src, emb_table, pos_table, ln_w, ln_b)
